# baseline (speedup 1.0000x reference)
import jax
import jax.numpy as jnp
from jax.experimental import pallas as pl
from jax.experimental.pallas import tpu as pltpu

D = 128
RADIUS = 0.5
R2 = RADIUS * RADIUS
SQRT_D = D ** 0.5
BI = 512
SLACK = 0.0625


def _vkm_body(bounds_ref, pts_ref, ptsT_ref, A_ref, out_re_ref, out_im_ref,
              epA_ref, epAb_ref):
    i = pl.program_id(0)

    @pl.when(i == 0)
    def _init():
        pts = pts_ref[...]
        pA = jnp.dot(pts.astype(jnp.bfloat16),
                     A_ref[...].astype(jnp.bfloat16),
                     preferred_element_type=jnp.float32)
        c = jnp.cos(pA)
        s = jnp.sin(pA)
        epA_ref[:, :D] = c
        epA_ref[:, D:] = s
        epAb_ref[:, :D] = c.astype(jnp.bfloat16)
        epAb_ref[:, D:] = s.astype(jnp.bfloat16)

    pts_i = pts_ref[pl.ds(i * BI, BI), :]
    yi = pts_i[:, 1:2]
    zi = pts_i[:, 2:3]
    sq_i = yi * yi + zi * zi
    pb_i = pts_i[:, 1:3].astype(jnp.bfloat16)

    def body(j, acc):
        pT = ptsT_ref[1:3, pl.ds(j * BI, BI)]
        sq_j = pT[0:1, :] * pT[0:1, :] + pT[1:2, :] * pT[1:2, :]
        cross = jnp.dot(pb_i, pT.astype(jnp.bfloat16),
                        preferred_element_type=jnp.float32)
        d2 = sq_i + sq_j - 2.0 * cross
        m = jnp.where(d2 < R2, 1.0, 0.0).astype(jnp.bfloat16)
        e = epAb_ref[pl.ds(j * BI, BI), :]
        return acc + jnp.dot(m, e, preferred_element_type=jnp.float32)

    jlo = bounds_ref[0, i]
    jhi = bounds_ref[1, i]
    acc = jax.lax.fori_loop(jlo, jhi, body,
                            jnp.zeros((BI, 2 * D), jnp.float32))

    ep = epA_ref[pl.ds(i * BI, BI), :]
    cr, ci = ep[:, :D], ep[:, D:]
    gr, gi = acc[:, :D], acc[:, D:]
    re = gr * cr + gi * ci
    im = gi * cr - gr * ci
    nrm = jnp.sqrt(jnp.sum(re * re + im * im, axis=1, keepdims=True))
    s = SQRT_D / nrm
    out_re_ref[...] = re * s
    out_im_ref[...] = im * s


def kernel(pts, A):
    n = pts.shape[0]
    npad = ((n + BI - 1) // BI) * BI
    pad = npad - n
    nb = npad // BI
    ptsf = pts.astype(jnp.float32)
    order = jnp.argsort(ptsf[:, 1])
    ptsp = ptsf[order]
    if pad:
        ptsp = jnp.concatenate(
            [ptsp, jnp.full((pad, 3), 1e4, jnp.float32)], axis=0)
    ptsT = ptsp.T
    ysb = ptsp[:, 1].reshape(nb, BI)
    bmin, bmax = ysb[:, 0], ysb[:, -1]
    th = RADIUS + SLACK
    jlo = jnp.sum(bmax[None, :] < (bmin - th)[:, None], axis=1)
    jhi = nb - jnp.sum(bmin[None, :] > (bmax + th)[:, None], axis=1)
    bounds = jnp.stack([jlo, jhi]).astype(jnp.int32)

    out_re, out_im = pl.pallas_call(
        _vkm_body,
        grid=(nb,),
        in_specs=[
            pl.BlockSpec(memory_space=pltpu.SMEM),
            pl.BlockSpec((npad, 3), lambda i: (0, 0)),
            pl.BlockSpec((3, npad), lambda i: (0, 0)),
            pl.BlockSpec((3, D), lambda i: (0, 0)),
        ],
        out_specs=[
            pl.BlockSpec((BI, D), lambda i: (i, 0)),
            pl.BlockSpec((BI, D), lambda i: (i, 0)),
        ],
        out_shape=[
            jax.ShapeDtypeStruct((npad, D), jnp.float32),
            jax.ShapeDtypeStruct((npad, D), jnp.float32),
        ],
        scratch_shapes=[
            pltpu.VMEM((npad, 2 * D), jnp.float32),
            pltpu.VMEM((npad, 2 * D), jnp.bfloat16),
        ],
        compiler_params=pltpu.CompilerParams(
            dimension_semantics=("arbitrary",)),
    )(bounds, ptsp, ptsT, A.astype(jnp.float32))
    inv = jnp.zeros((n,), jnp.int32).at[order].set(jnp.arange(n, dtype=jnp.int32))
    return (out_re[:n] + 1j * out_im[:n]).astype(jnp.complex64)[inv]


# speedup vs baseline: 1.0007x; 1.0007x over previous
import jax
import jax.numpy as jnp
from jax.experimental import pallas as pl
from jax.experimental.pallas import tpu as pltpu

D = 128
RADIUS = 0.5
R2 = RADIUS * RADIUS
SQRT_D = D ** 0.5
BI = 512
SLACK = 0.0625


def _vkm_body(bounds_ref, pts_ref, ptsT_ref, A_ref, out_re_ref, out_im_ref,
              epA_ref, epAb_ref):
    i = pl.program_id(0)

    @pl.when(i == 0)
    def _init():
        pts = pts_ref[...]
        pA = jnp.dot(pts.astype(jnp.bfloat16),
                     A_ref[...].astype(jnp.bfloat16),
                     preferred_element_type=jnp.float32)
        c = jnp.cos(pA)
        s = jnp.sin(pA)
        epA_ref[:, :D] = c
        epA_ref[:, D:] = s
        epAb_ref[:, :D] = c.astype(jnp.bfloat16)
        epAb_ref[:, D:] = s.astype(jnp.bfloat16)

    pts_i = pts_ref[pl.ds(i * BI, BI), :]
    yi = pts_i[:, 1:2]
    zi = pts_i[:, 2:3]
    sq_i = yi * yi + zi * zi
    pb_i = pts_i[:, 1:3].astype(jnp.bfloat16)

    def body(j, acc):
        pT = ptsT_ref[1:3, pl.ds(j * BI, BI)]
        sq_j = pT[0:1, :] * pT[0:1, :] + pT[1:2, :] * pT[1:2, :]
        cross = jnp.dot(pb_i, pT.astype(jnp.bfloat16),
                        preferred_element_type=jnp.float32)
        d2 = sq_i + sq_j - 2.0 * cross
        m = jnp.where(d2 < R2, 1.0, 0.0).astype(jnp.bfloat16)
        e = epAb_ref[pl.ds(j * BI, BI), :]
        return acc + jnp.dot(m, e, preferred_element_type=jnp.float32)

    jlo = bounds_ref[0, i]
    jhi = bounds_ref[1, i]
    acc = jax.lax.fori_loop(jlo, jhi, body,
                            jnp.zeros((BI, 2 * D), jnp.float32))

    ep = epA_ref[pl.ds(i * BI, BI), :]
    cr, ci = ep[:, :D], ep[:, D:]
    gr, gi = acc[:, :D], acc[:, D:]
    re = gr * cr + gi * ci
    im = gi * cr - gr * ci
    nrm = jnp.sqrt(jnp.sum(re * re + im * im, axis=1, keepdims=True))
    s = SQRT_D / nrm
    out_re_ref[...] = re * s
    out_im_ref[...] = im * s


def kernel(pts, A):
    n = pts.shape[0]
    npad = ((n + BI - 1) // BI) * BI
    pad = npad - n
    nb = npad // BI
    ptsf = pts.astype(jnp.float32)
    order = jnp.argsort(ptsf[:, 1], stable=False)
    ptsp = ptsf[order]
    if pad:
        ptsp = jnp.concatenate(
            [ptsp, jnp.full((pad, 3), 1e4, jnp.float32)], axis=0)
    ptsT = ptsp.T
    ysb = ptsp[:, 1].reshape(nb, BI)
    bmin, bmax = ysb[:, 0], ysb[:, -1]
    th = RADIUS + SLACK
    jlo = jnp.sum(bmax[None, :] < (bmin - th)[:, None], axis=1)
    jhi = nb - jnp.sum(bmin[None, :] > (bmax + th)[:, None], axis=1)
    bounds = jnp.stack([jlo, jhi]).astype(jnp.int32)

    out_re, out_im = pl.pallas_call(
        _vkm_body,
        grid=(nb,),
        in_specs=[
            pl.BlockSpec(memory_space=pltpu.SMEM),
            pl.BlockSpec((npad, 3), lambda i: (0, 0)),
            pl.BlockSpec((3, npad), lambda i: (0, 0)),
            pl.BlockSpec((3, D), lambda i: (0, 0)),
        ],
        out_specs=[
            pl.BlockSpec((BI, D), lambda i: (i, 0)),
            pl.BlockSpec((BI, D), lambda i: (i, 0)),
        ],
        out_shape=[
            jax.ShapeDtypeStruct((npad, D), jnp.float32),
            jax.ShapeDtypeStruct((npad, D), jnp.float32),
        ],
        scratch_shapes=[
            pltpu.VMEM((npad, 2 * D), jnp.float32),
            pltpu.VMEM((npad, 2 * D), jnp.bfloat16),
        ],
        compiler_params=pltpu.CompilerParams(
            dimension_semantics=("arbitrary",)),
    )(bounds, ptsp, ptsT, A.astype(jnp.float32))
    inv = jnp.zeros((n,), jnp.int32).at[order].set(jnp.arange(n, dtype=jnp.int32))
    return (out_re[:n] + 1j * out_im[:n]).astype(jnp.complex64)[inv]


# speedup vs baseline: 1.0865x; 1.0858x over previous
import jax
import jax.numpy as jnp
from jax.experimental import pallas as pl
from jax.experimental.pallas import tpu as pltpu

D = 128
RADIUS = 0.5
R2 = RADIUS * RADIUS
SQRT_D = D ** 0.5
BI = 512


def _vkm_body(pts_ref, ptsT_ref, A_ref, out_re_ref, out_im_ref,
              epA_ref, epAr_ref):
    i = pl.program_id(0)
    npad = pts_ref.shape[0]
    nblocks = npad // BI

    @pl.when(i == 0)
    def _init():
        pts = pts_ref[...]
        pA = jnp.dot(pts.astype(jnp.bfloat16),
                     A_ref[...].astype(jnp.bfloat16),
                     preferred_element_type=jnp.float32)
        c = jnp.cos(pA)
        s = jnp.sin(pA)
        epA_ref[:, :D] = c
        epA_ref[:, D:] = s
        epAr_ref[:, :D] = c.astype(jnp.bfloat16).astype(jnp.float32)
        epAr_ref[:, D:] = s.astype(jnp.bfloat16).astype(jnp.float32)

    pts_i = pts_ref[pl.ds(i * BI, BI), :]
    yi = pts_i[:, 1:2]
    zi = pts_i[:, 2:3]
    sq_i = yi * yi + zi * zi
    pb_i = pts_i[:, 1:3].astype(jnp.bfloat16)

    def body(j, acc):
        pT = ptsT_ref[1:3, pl.ds(j * BI, BI)]
        sq_j = pT[0:1, :] * pT[0:1, :] + pT[1:2, :] * pT[1:2, :]
        cross = jnp.dot(pb_i, pT.astype(jnp.bfloat16),
                        preferred_element_type=jnp.float32)
        d2 = sq_i + sq_j - 2.0 * cross
        m = jnp.where(d2 < R2, 1.0, 0.0)
        e = epAr_ref[pl.ds(j * BI, BI), :]
        return acc + jnp.dot(m, e, preferred_element_type=jnp.float32)

    acc = jax.lax.fori_loop(0, nblocks, body,
                            jnp.zeros((BI, 2 * D), jnp.float32))

    ep = epA_ref[pl.ds(i * BI, BI), :]
    cr, ci = ep[:, :D], ep[:, D:]
    gr, gi = acc[:, :D], acc[:, D:]
    re = gr * cr + gi * ci
    im = gi * cr - gr * ci
    nrm = jnp.sqrt(jnp.sum(re * re + im * im, axis=1, keepdims=True))
    s = SQRT_D / nrm
    out_re_ref[...] = re * s
    out_im_ref[...] = im * s


def kernel(pts, A):
    n = pts.shape[0]
    npad = ((n + BI - 1) // BI) * BI
    pad = npad - n
    ptsp = pts.astype(jnp.float32)
    if pad:
        ptsp = jnp.concatenate(
            [ptsp, jnp.full((pad, 3), 1e4, jnp.float32)], axis=0)
    ptsT = ptsp.T
    out_re, out_im = pl.pallas_call(
        _vkm_body,
        grid=(npad // BI,),
        in_specs=[
            pl.BlockSpec((npad, 3), lambda i: (0, 0)),
            pl.BlockSpec((3, npad), lambda i: (0, 0)),
            pl.BlockSpec((3, D), lambda i: (0, 0)),
        ],
        out_specs=[
            pl.BlockSpec((BI, D), lambda i: (i, 0)),
            pl.BlockSpec((BI, D), lambda i: (i, 0)),
        ],
        out_shape=[
            jax.ShapeDtypeStruct((npad, D), jnp.float32),
            jax.ShapeDtypeStruct((npad, D), jnp.float32),
        ],
        scratch_shapes=[
            pltpu.VMEM((npad, 2 * D), jnp.float32),
            pltpu.VMEM((npad, 2 * D), jnp.float32),
        ],
        compiler_params=pltpu.CompilerParams(
            dimension_semantics=("arbitrary",)),
    )(ptsp, ptsT, A.astype(jnp.float32))
    return (out_re[:n] + 1j * out_im[:n]).astype(jnp.complex64)
